# Initial kernel scaffold; baseline (speedup 1.0000x reference)
import jax
import jax.numpy as jnp
from jax.experimental import pallas as pl


def kernel(X1, X2, A, a, gamma, graph):
    raise NotImplementedError("write your pallas kernel here")


# speedup vs baseline: 25.2014x; 25.2014x over previous
import jax
import jax.numpy as jnp
from jax import lax
from jax.experimental import pallas as pl
from jax.experimental.pallas import tpu as pltpu

NSP = 32


def _wdk_kernel(x1t_ref, x2t_ref, a_ref, out_ref, kacc_ref, k1_ref, k2_ref):
    Lx, n1 = x1t_ref.shape
    n2 = x2t_ref.shape[1]
    A = a_ref[:]
    S = lax.dot_general(A, A, (((1,), (1,)), ((), ())),
                        preferred_element_type=jnp.float32)
    dcol = jnp.sum(A * A, axis=1, keepdims=True)

    iota1 = lax.broadcasted_iota(jnp.int32, (NSP, n1), 0)
    iota2 = lax.broadcasted_iota(jnp.int32, (NSP, n2), 0)

    def columns(i):
        r1 = x1t_ref[pl.ds(i, 1), :]
        r2 = x2t_ref[pl.ds(i, 1), :]
        oh1 = (iota1 == r1).astype(jnp.float32)
        oh2 = (iota2 == r2).astype(jnp.float32)
        r1s = lax.dot_general(S, oh1, (((0,), (0,)), ((), ())),
                              preferred_element_type=jnp.float32)
        t = lax.dot_general(r1s, oh2, (((0,), (0,)), ((), ())),
                            preferred_element_type=jnp.float32)
        d1 = jnp.sum(oh1 * dcol, axis=0, keepdims=True)
        d2 = jnp.sum(oh2 * dcol, axis=0, keepdims=True)
        return t, d1, d2

    t0, d10, d20 = columns(0)
    kacc_ref[:] = jnp.zeros((n1, n2), jnp.float32)
    k1_ref[:] = jnp.zeros((1, n1), jnp.float32)
    k2_ref[:] = jnp.zeros((1, n2), jnp.float32)

    def body(i, carry):
        t_prev, d1_prev, d2_prev = carry
        t, d1, d2 = columns(i)
        kacc_ref[:] += t_prev * t
        k1_ref[:] += d1_prev * d1
        k2_ref[:] += d2_prev * d2
        return t, d1, d2

    lax.fori_loop(1, Lx, body, (t0, d10, d20))

    k0 = 2.0 * kacc_ref[:]
    k1c = jnp.transpose(2.0 * k1_ref[:])
    k2r = 2.0 * k2_ref[:]
    out_ref[:] = k0 / jnp.sqrt(k1c) / jnp.sqrt(k2r)


def kernel(X1, X2, A, a, gamma, graph):
    n1, Lx = X1.shape
    n2 = X2.shape[0]
    ns, d = A.shape
    X1T = X1.T
    X2T = X2.T
    Apad = jnp.zeros((NSP, d), jnp.float32).at[:ns].set(A)
    ratio = pl.pallas_call(
        _wdk_kernel,
        out_shape=jax.ShapeDtypeStruct((n1, n2), jnp.float32),
        scratch_shapes=[
            pltpu.VMEM((n1, n2), jnp.float32),
            pltpu.VMEM((1, n1), jnp.float32),
            pltpu.VMEM((1, n2), jnp.float32),
        ],
    )(X1T, X2T, Apad)
    return (a**2) * ratio**gamma


# speedup vs baseline: 98.2856x; 3.9000x over previous
import jax
import jax.numpy as jnp
from jax import lax
from jax.experimental import pallas as pl
from jax.experimental.pallas import tpu as pltpu

NSP = 32


def _wdk_kernel(x1f_ref, x2f_ref, a_ref, out_ref, oh1_s, oh2_s, r1s_s, d1_s, d2_s):
    LN = x1f_ref.shape[1]
    n1 = out_ref.shape[0]
    n2 = out_ref.shape[1]
    Lx = LN // n1

    A = a_ref[:]
    S = lax.dot_general(A, A, (((1,), (1,)), ((), ())),
                        preferred_element_type=jnp.float32)
    drow = jnp.transpose(jnp.sum(A * A, axis=1, keepdims=True))

    ci = lax.broadcasted_iota(jnp.int32, (NSP, LN), 0)
    oh1_s[:] = (ci == x1f_ref[:]).astype(jnp.float32)
    oh2_s[:] = (ci == x2f_ref[:]).astype(jnp.float32)
    r1s_s[:] = lax.dot_general(S, oh1_s[:], (((0,), (0,)), ((), ())),
                               preferred_element_type=jnp.float32)
    d1_s[:] = jnp.dot(drow, oh1_s[:], preferred_element_type=jnp.float32)
    d2_s[:] = jnp.dot(drow, oh2_s[:], preferred_element_type=jnp.float32)

    def t_at(i):
        r1s_i = r1s_s[:, i * n1:(i + 1) * n1]
        oh2_i = oh2_s[:, i * n2:(i + 1) * n2]
        return lax.dot_general(r1s_i, oh2_i, (((0,), (0,)), ((), ())),
                               preferred_element_type=jnp.float32)

    acc0 = jnp.zeros((n1, n2), jnp.float32)
    acc1 = jnp.zeros((n1, n2), jnp.float32)
    t_prev = t_at(0)
    for i in range(1, Lx):
        t = t_at(i)
        if i % 2:
            acc0 = acc0 + t_prev * t
        else:
            acc1 = acc1 + t_prev * t
        t_prev = t
    k0 = 2.0 * (acc0 + acc1)

    k1 = jnp.zeros((1, n1), jnp.float32)
    k2 = jnp.zeros((1, n2), jnp.float32)
    for i in range(Lx - 1):
        k1 = k1 + d1_s[:, i * n1:(i + 1) * n1] * d1_s[:, (i + 1) * n1:(i + 2) * n1]
        k2 = k2 + d2_s[:, i * n2:(i + 1) * n2] * d2_s[:, (i + 1) * n2:(i + 2) * n2]

    k1c = jnp.transpose(2.0 * k1)
    k2r = 2.0 * k2
    out_ref[:] = k0 / jnp.sqrt(k1c) / jnp.sqrt(k2r)


def kernel(X1, X2, A, a, gamma, graph):
    n1, Lx = X1.shape
    n2 = X2.shape[0]
    ns, d = A.shape
    X1f = X1.T.reshape(1, Lx * n1)
    X2f = X2.T.reshape(1, Lx * n2)
    Apad = jnp.zeros((NSP, d), jnp.float32).at[:ns].set(A)
    ratio = pl.pallas_call(
        _wdk_kernel,
        out_shape=jax.ShapeDtypeStruct((n1, n2), jnp.float32),
        scratch_shapes=[
            pltpu.VMEM((NSP, Lx * n1), jnp.float32),
            pltpu.VMEM((NSP, Lx * n2), jnp.float32),
            pltpu.VMEM((NSP, Lx * n1), jnp.float32),
            pltpu.VMEM((1, Lx * n1), jnp.float32),
            pltpu.VMEM((1, Lx * n2), jnp.float32),
        ],
    )(X1f, X2f, Apad)
    return (a**2) * ratio**gamma


# speedup vs baseline: 140.1601x; 1.4260x over previous
import jax
import jax.numpy as jnp
from jax import lax
from jax.experimental import pallas as pl
from jax.experimental.pallas import tpu as pltpu


def _wdk_kernel(x1_ref, x2_ref, a_ref, as_ref, gs_ref, out_ref,
                oh1_s, oh2_s, r1s_s, d1_s, d2_s, x1t_s, x2t_s):
    n1, Lx = x1_ref.shape
    n2 = x2_ref.shape[0]
    ns = a_ref.shape[0]

    A = a_ref[:]
    S = lax.dot_general(A, A, (((1,), (1,)), ((), ())),
                        preferred_element_type=jnp.float32)
    drow = jnp.transpose(jnp.sum(A * A, axis=1, keepdims=True))

    x1t_s[:] = jnp.transpose(x1_ref[:].astype(jnp.float32))
    x2t_s[:] = jnp.transpose(x2_ref[:].astype(jnp.float32))

    iota1 = lax.broadcasted_iota(jnp.int32, (ns, n1), 0).astype(jnp.float32)
    iota2 = lax.broadcasted_iota(jnp.int32, (ns, n2), 0).astype(jnp.float32)
    for i in range(Lx):
        oh1_s[:, i * n1:(i + 1) * n1] = (iota1 == x1t_s[i:i + 1, :]).astype(jnp.float32)
        oh2_s[:, i * n2:(i + 1) * n2] = (iota2 == x2t_s[i:i + 1, :]).astype(jnp.float32)

    r1s_s[:] = lax.dot_general(S, oh1_s[:], (((0,), (0,)), ((), ())),
                               preferred_element_type=jnp.float32)
    d1_s[:] = jnp.dot(drow, oh1_s[:], preferred_element_type=jnp.float32)
    d2_s[:] = jnp.dot(drow, oh2_s[:], preferred_element_type=jnp.float32)

    def t_at(i):
        r1s_i = r1s_s[:, i * n1:(i + 1) * n1]
        oh2_i = oh2_s[:, i * n2:(i + 1) * n2]
        return lax.dot_general(r1s_i, oh2_i, (((0,), (0,)), ((), ())),
                               preferred_element_type=jnp.float32)

    acc0 = jnp.zeros((n1, n2), jnp.float32)
    acc1 = jnp.zeros((n1, n2), jnp.float32)
    t_prev = t_at(0)
    for i in range(1, Lx):
        t = t_at(i)
        if i % 2:
            acc0 = acc0 + t_prev * t
        else:
            acc1 = acc1 + t_prev * t
        t_prev = t
    k0 = 2.0 * (acc0 + acc1)

    k1 = jnp.zeros((1, n1), jnp.float32)
    k2 = jnp.zeros((1, n2), jnp.float32)
    for i in range(Lx - 1):
        k1 = k1 + d1_s[:, i * n1:(i + 1) * n1] * d1_s[:, (i + 1) * n1:(i + 2) * n1]
        k2 = k2 + d2_s[:, i * n2:(i + 1) * n2] * d2_s[:, (i + 1) * n2:(i + 2) * n2]

    k1c = jnp.transpose(2.0 * k1)
    k2r = 2.0 * k2
    ratio = k0 / jnp.sqrt(k1c) / jnp.sqrt(k2r)

    av = as_ref[0, 0]
    gv = gs_ref[0, 0]
    powed = jnp.where(gv == jnp.float32(1.0), ratio,
                      jnp.exp(gv * jnp.log(ratio)))
    out_ref[:] = (av * av) * powed


def kernel(X1, X2, A, a, gamma, graph):
    n1, Lx = X1.shape
    n2 = X2.shape[0]
    ns, d = A.shape
    return pl.pallas_call(
        _wdk_kernel,
        out_shape=jax.ShapeDtypeStruct((n1, n2), jnp.float32),
        in_specs=[
            pl.BlockSpec(memory_space=pltpu.VMEM),
            pl.BlockSpec(memory_space=pltpu.VMEM),
            pl.BlockSpec(memory_space=pltpu.VMEM),
            pl.BlockSpec(memory_space=pltpu.SMEM),
            pl.BlockSpec(memory_space=pltpu.SMEM),
        ],
        scratch_shapes=[
            pltpu.VMEM((ns, Lx * n1), jnp.float32),
            pltpu.VMEM((ns, Lx * n2), jnp.float32),
            pltpu.VMEM((ns, Lx * n1), jnp.float32),
            pltpu.VMEM((1, Lx * n1), jnp.float32),
            pltpu.VMEM((1, Lx * n2), jnp.float32),
            pltpu.VMEM((Lx, n1), jnp.float32),
            pltpu.VMEM((Lx, n2), jnp.float32),
        ],
    )(X1, X2, A.astype(jnp.float32), a.reshape(1, 1), gamma.reshape(1, 1))
